# baseline (speedup 1.0000x reference)
import functools

import jax
import jax.numpy as jnp
from jax import lax
from jax.experimental import pallas as pl
from jax.experimental.pallas import tpu as pltpu
from jax.experimental.pallas import tpu_sc as plsc

_B, _N, _K = 2, 512, 32
_D0, _D1, _H = 128, 16, 32
_EIN = 2 * _D0 + _D1 + 1
_F1 = 2 * _EIN
_F2 = 4 * _H
_GW = 128
_E = _B * _N * _K
_LN_EPS = 1e-5
_HN_EPS = 1e-8

_BN = 64
_R = _BN * _K

_NW = 32
_EPW = _E // _NW
_CH = 128
_NCH = _EPW // _CH


def _gather_rows(tbl, fidx):
    mesh = plsc.VectorSubcoreMesh(core_axis_name="c", subcore_axis_name="s")

    @functools.partial(
        pl.kernel,
        out_type=jax.ShapeDtypeStruct((_E, _GW), jnp.int32),
        mesh=mesh,
        scratch_types=[
            pltpu.VMEM((_CH,), jnp.int32),
            pltpu.VMEM((_CH,), jnp.int32),
            pltpu.VMEM((_CH, _GW), jnp.int32),
            pltpu.VMEM((_CH, _GW), jnp.int32),
            pltpu.SemaphoreType.DMA,
            pltpu.SemaphoreType.DMA,
        ],
    )
    def gk(tbl_hbm, idx_hbm, out_hbm, idx0, idx1, buf0, buf1, sem0, sem1):
        wid = lax.axis_index("s") * 2 + lax.axis_index("c")
        base = wid * _EPW
        idxs = (idx0, idx1)
        bufs = (buf0, buf1)
        sems = (sem0, sem1)
        pltpu.sync_copy(idx_hbm.at[pl.ds(base, _CH)], idx0)
        pltpu.async_copy(tbl_hbm.at[idx0], buf0, sem0)
        for c in range(_NCH):
            cur = c % 2
            nxt = 1 - cur
            off = base + c * _CH
            if c + 1 < _NCH:
                noff = off + _CH
                pltpu.sync_copy(idx_hbm.at[pl.ds(noff, _CH)], idxs[nxt])
                pltpu.async_copy(tbl_hbm.at[idxs[nxt]], bufs[nxt], sems[nxt])
            pltpu.make_async_copy(tbl_hbm.at[idxs[cur]], bufs[cur], sems[cur]).wait()
            pltpu.sync_copy(bufs[cur], out_hbm.at[pl.ds(off, _CH)])

    return gk(tbl, fidx)


def _silu(x):
    return x * (0.5 * jnp.tanh(0.5 * x) + 0.5)


def _dotb(a, b):
    return lax.dot_general(a, b, (((1,), (0,)), ((), ())),
                           preferred_element_type=jnp.float32)


def _lo_f32(w):
    return lax.bitcast_convert_type(w << 16, jnp.float32)


def _hi_f32(w):
    return lax.bitcast_convert_type(w & jnp.int32(-65536), jnp.float32)


def _tc_body(nodes_r, ht48_r, g_r, aux_r, sel_r,
             wa_r, wb_r, wc_r, wdp_r, b1_r, w2_r, b2_r,
             hw1_r, hb1_r, hw2_r, hb2_r,
             nw1a_r, nw1b_r, nb1_r, nw2_r, nb2_r,
             gw_r, gb_r, lns_r, lnb_r, hns_r, hnb_r,
             on_r, ofx_r, ofy_r, ofz_r):
    bf = jnp.bfloat16
    nodes = nodes_r[0]
    g = g_r[0]
    aux = aux_r[0]
    mk_e = aux[:, 1:2]

    nw = g[:, :_D0 // 2]
    cw = g[:, _D0 // 2:_D0 // 2 + 24]
    nj_lo = _lo_f32(nw).astype(bf)
    nj_hi = _hi_f32(nw).astype(bf)
    xyz_j = jnp.concatenate([_lo_f32(cw), _hi_f32(cw)], axis=-1)
    xyz_j = xyz_j.reshape(_BN, _K, 3 * _D1)
    ht48 = ht48_r[0]

    rel = ht48[:, None, :] - xyz_j
    r2 = rel * rel
    nsq = r2[:, :, :_D1] + r2[:, :, _D1:2 * _D1] + r2[:, :, 2 * _D1:]
    norm = jnp.sqrt(nsq)

    pre_a = (_dotb(nodes.astype(bf), wa_r[:]) + b1_r[:]).astype(bf)
    catx = jnp.concatenate(
        [nj_lo, nj_hi, sel_r[:], norm.reshape(_R, _D1).astype(bf),
         aux.astype(bf)], axis=-1)
    wcat = jnp.concatenate(
        [wb_r[:], pre_a, wc_r[:], wdp_r[:]], axis=0)
    h1 = _silu(_dotb(catx, wcat)).astype(bf)
    m = _silu(_dotb(h1, w2_r[:]) + b2_r[:])

    t = _silu(_dotb(m.astype(bf), hw1_r[:]) + hb1_r[:])
    htw = _dotb(t.astype(bf), hw2_r[:]) + hb2_r[:]

    hns = hns_r[:].reshape(1, 1, _D1)
    hnb = hnb_r[:].reshape(1, 1, _D1)
    coeff = (norm * hns + hnb) / jnp.maximum(norm, _HN_EPS)
    w3 = htw.reshape(_BN, _K, _D1) * coeff
    w3_48 = jnp.concatenate([w3, w3, w3], axis=-1)
    prod = (rel * w3_48).reshape(_R, 3 * _D1).astype(bf)
    upd48 = lax.dot_general(sel_r[:], prod, (((0,), (0,)), ((), ())),
                            preferred_element_type=jnp.float32)

    mmk = (m * mk_e).astype(bf)
    mi = lax.dot_general(sel_r[:], mmk, (((0,), (0,)), ((), ())),
                         preferred_element_type=jnp.float32)

    mu = jnp.mean(nodes, axis=-1, keepdims=True)
    var = jnp.mean((nodes - mu) ** 2, axis=-1, keepdims=True)
    normed = (nodes - mu) * lax.rsqrt(var + _LN_EPS) * lns_r[:] + lnb_r[:]
    h = _silu(_dotb(normed.astype(bf), nw1a_r[:])
              + _dotb(mi.astype(bf), nw1b_r[:]) + nb1_r[:])
    node_out = _dotb(h.astype(bf), nw2_r[:]) + nb2_r[:] + nodes
    gate = 0.5 * jnp.tanh(
        0.5 * (_dotb(node_out.astype(bf), gw_r[:]) + gb_r[:])) + 0.5

    nf = ht48 + upd48
    on_r[0] = node_out
    ofx_r[0] = nf[:, :_D1] * gate
    ofy_r[0] = nf[:, _D1:2 * _D1] * gate
    ofz_r[0] = nf[:, 2 * _D1:] * gate


def _pack_halves(x):
    u = lax.bitcast_convert_type(
        x.astype(jnp.bfloat16), jnp.uint16).astype(jnp.uint32)
    n = x.shape[-1] // 2
    return lax.bitcast_convert_type(
        u[..., :n] | (u[..., n:] << 16), jnp.int32)


def kernel(node_feats, htype1, rel_dist, neighbor_indices, neighbor_masks,
           ln_scale, ln_bias, e_w1, e_b1, e_w2, e_b2, hn_scale, hn_bias,
           gate_w, gate_b, ht_w1, ht_b1, ht_w2, ht_b2, n_w1, n_b1, n_w2, n_b2):
    f32 = jnp.float32
    bf = jnp.bfloat16
    nodes = node_feats[..., 0]
    ht48 = jnp.concatenate(
        [htype1[..., 0], htype1[..., 1], htype1[..., 2]],
        axis=-1)

    tblw = jnp.concatenate(
        [_pack_halves(nodes), _pack_halves(ht48),
         jnp.zeros((_B, _N, _GW - _D0 // 2 - 3 * _D1 // 2), jnp.int32)],
        axis=-1).reshape(_B * _N, _GW)
    fidx = (jnp.arange(_B, dtype=jnp.int32)[:, None, None] * _N
            + neighbor_indices.astype(jnp.int32)).reshape(_E)
    g = _gather_rows(tblw, fidx).reshape(_B, _N * _K, _GW)

    lane16 = jnp.arange(_D1, dtype=jnp.int32)
    aux = (rel_dist[..., None] * (lane16 == 0)
           + neighbor_masks[..., None].astype(f32) * (lane16 == 1)
           ).reshape(_B, _N * _K, _D1)

    wa = e_w1[:_D0].astype(bf)
    wbm = e_w1[_D0:2 * _D0]
    wb = wbm.astype(bf)
    wc = e_w1[2 * _D0:2 * _D0 + _D1].astype(bf)
    wdp = jnp.concatenate(
        [e_w1[2 * _D0 + _D1:], jnp.zeros((_D1 - 1, _F1), f32)],
        axis=0).astype(bf)
    sel = (jnp.arange(_R, dtype=jnp.int32)[:, None] // _K
           == jnp.arange(_BN, dtype=jnp.int32)[None, :]).astype(bf)
    nw1a = n_w1[:_D0].astype(bf)
    nw1b = n_w1[_D0:].astype(bf)
    row = lambda v: v.reshape(1, -1)

    grid = (_B, _N // _BN)
    node_spec = pl.BlockSpec((1, _BN, _D0), lambda b, i: (b, i, 0))
    d1_spec = pl.BlockSpec((1, _BN, _D1), lambda b, i: (b, i, 0))
    ht_spec = pl.BlockSpec((1, _BN, 3 * _D1), lambda b, i: (b, i, 0))
    g_spec = pl.BlockSpec((1, _R, _GW), lambda b, i: (b, i, 0))
    aux_spec = pl.BlockSpec((1, _R, _D1), lambda b, i: (b, i, 0))
    _full = lambda shape: pl.BlockSpec(shape, lambda b, i: (0, 0))

    on, ofx, ofy, ofz = pl.pallas_call(
        _tc_body,
        grid=grid,
        in_specs=[
            node_spec, ht_spec, g_spec, aux_spec,
            _full((_R, _BN)),
            _full((_D0, _F1)), _full((_D0, _F1)), _full((_D1, _F1)),
            _full((_D1, _F1)), _full((1, _F1)), _full((_F1, _H)),
            _full((1, _H)),
            _full((_H, _F2)), _full((1, _F2)), _full((_F2, _D1)),
            _full((1, _D1)),
            _full((_D0, 2 * _D0)), _full((_H, 2 * _D0)), _full((1, 2 * _D0)),
            _full((2 * _D0, _D0)), _full((1, _D0)),
            _full((_D0, _D1)), _full((1, _D1)), _full((1, _D0)),
            _full((1, _D0)), _full((1, _D1)), _full((1, _D1)),
        ],
        out_specs=[node_spec, d1_spec, d1_spec, d1_spec],
        out_shape=[
            jax.ShapeDtypeStruct((_B, _N, _D0), f32),
            jax.ShapeDtypeStruct((_B, _N, _D1), f32),
            jax.ShapeDtypeStruct((_B, _N, _D1), f32),
            jax.ShapeDtypeStruct((_B, _N, _D1), f32),
        ],
    )(nodes, ht48, g, aux, sel,
      wa, wb, wc, wdp, row(e_b1), e_w2.astype(bf), row(e_b2),
      ht_w1.astype(bf), row(ht_b1), ht_w2.astype(bf), row(ht_b2),
      nw1a, nw1b, row(n_b1), n_w2.astype(bf), row(n_b2),
      gate_w.astype(bf), row(gate_b), row(ln_scale), row(ln_bias),
      hn_scale.reshape(1, _D1), hn_bias.reshape(1, _D1))

    node_out = on[..., None]
    feat1 = jnp.stack([ofx, ofy, ofz], axis=-1)
    return node_out, feat1


# speedup vs baseline: 1.0874x; 1.0874x over previous
import functools

import jax
import jax.numpy as jnp
from jax import lax
from jax.experimental import pallas as pl
from jax.experimental.pallas import tpu as pltpu
from jax.experimental.pallas import tpu_sc as plsc

_B, _N, _K = 2, 512, 32
_D0, _D1, _H = 128, 16, 32
_EIN = 2 * _D0 + _D1 + 1
_F1 = 2 * _EIN
_F2 = 4 * _H
_GW = 128
_E = _B * _N * _K
_LN_EPS = 1e-5
_HN_EPS = 1e-8

_BN = 64
_R = _BN * _K

_NW = 32
_EPW = _E // _NW
_CH = 128
_NCH = _EPW // _CH


def _gather_rows(tbl, fidx):
    mesh = plsc.VectorSubcoreMesh(core_axis_name="c", subcore_axis_name="s")

    @functools.partial(
        pl.kernel,
        out_type=jax.ShapeDtypeStruct((_E, _GW), jnp.int32),
        mesh=mesh,
        scratch_types=[
            pltpu.VMEM((_CH,), jnp.int32),
            pltpu.VMEM((_CH,), jnp.int32),
            pltpu.VMEM((_CH, _GW), jnp.int32),
            pltpu.VMEM((_CH, _GW), jnp.int32),
            pltpu.SemaphoreType.DMA,
            pltpu.SemaphoreType.DMA,
        ],
    )
    def gk(tbl_hbm, idx_hbm, out_hbm, idx0, idx1, buf0, buf1, sem0, sem1):
        wid = lax.axis_index("s") * 2 + lax.axis_index("c")
        base = wid * _EPW
        idxs = (idx0, idx1)
        bufs = (buf0, buf1)
        sems = (sem0, sem1)
        pltpu.sync_copy(idx_hbm.at[pl.ds(base, _CH)], idx0)
        pltpu.async_copy(tbl_hbm.at[idx0], buf0, sem0)
        for c in range(_NCH):
            cur = c % 2
            nxt = 1 - cur
            off = base + c * _CH
            if c + 1 < _NCH:
                noff = off + _CH
                pltpu.sync_copy(idx_hbm.at[pl.ds(noff, _CH)], idxs[nxt])
                pltpu.async_copy(tbl_hbm.at[idxs[nxt]], bufs[nxt], sems[nxt])
            pltpu.make_async_copy(tbl_hbm.at[idxs[cur]], bufs[cur], sems[cur]).wait()
            pltpu.sync_copy(bufs[cur], out_hbm.at[pl.ds(off, _CH)])

    return gk(tbl, fidx)


def _silu(x):
    return x * (0.5 * jnp.tanh(0.5 * x) + 0.5)


def _dotb(a, b):
    return lax.dot_general(a, b, (((1,), (0,)), ((), ())),
                           preferred_element_type=jnp.float32)


def _lo_f32(w):
    return lax.bitcast_convert_type(w << 16, jnp.float32)


def _hi_f32(w):
    return lax.bitcast_convert_type(w & jnp.int32(-65536), jnp.float32)


def _tc_body(nodes_r, ht48_r, g_r, aux_r, sel_r,
             wa_r, wb_r, wc_r, wdp_r, b1_r, w2_r, b2_r,
             hw1_r, hb1_r, hw2_r, hb2_r,
             nw1a_r, nw1b_r, nb1_r, nw2_r, nb2_r,
             gw_r, gb_r, lns_r, lnb_r, hns_r, hnb_r,
             on_r, ofx_r, ofy_r, ofz_r):
    bf = jnp.bfloat16
    nodes = nodes_r[0]
    g = g_r[0]
    aux = aux_r[0]
    mk_e = aux[:, 1:2]

    nw = g[:, :_D0 // 2]
    cw = g[:, _D0 // 2:_D0 // 2 + 24]
    nj_lo = _lo_f32(nw).astype(bf)
    nj_hi = _hi_f32(nw).astype(bf)
    xyz_j = jnp.concatenate([_lo_f32(cw), _hi_f32(cw)], axis=-1)
    xyz_j = xyz_j.reshape(_BN, _K, 3 * _D1)
    ht48 = ht48_r[0]

    rel = ht48[:, None, :] - xyz_j
    r2 = rel * rel
    nsq = r2[:, :, :_D1] + r2[:, :, _D1:2 * _D1] + r2[:, :, 2 * _D1:]
    norm = jnp.sqrt(nsq)

    pre_a = (_dotb(nodes.astype(bf), wa_r[:]) + b1_r[:]).astype(bf)
    catx = jnp.concatenate(
        [nj_lo, nj_hi, sel_r[:], norm.reshape(_R, _D1).astype(bf),
         aux.astype(bf)], axis=-1)
    wcat = jnp.concatenate(
        [wb_r[:], pre_a, wc_r[:], wdp_r[:]], axis=0)
    h1 = _silu(_dotb(catx, wcat)).astype(bf)
    m = _silu(_dotb(h1, w2_r[:]) + b2_r[:])

    t = _silu(_dotb(m.astype(bf), hw1_r[:]) + hb1_r[:])
    htw = _dotb(t.astype(bf), hw2_r[:]) + hb2_r[:]

    hns = hns_r[:].reshape(1, 1, _D1)
    hnb = hnb_r[:].reshape(1, 1, _D1)
    coeff = (norm * hns + hnb) / jnp.maximum(norm, _HN_EPS)
    w3 = htw.reshape(_BN, _K, _D1) * coeff
    w3_48 = jnp.concatenate([w3, w3, w3], axis=-1)
    upd48 = jnp.sum(rel * w3_48, axis=1)

    mmk = (m * mk_e).astype(bf)
    mi = lax.dot_general(sel_r[:], mmk, (((0,), (0,)), ((), ())),
                         preferred_element_type=jnp.float32)

    mu = jnp.mean(nodes, axis=-1, keepdims=True)
    var = jnp.mean((nodes - mu) ** 2, axis=-1, keepdims=True)
    normed = (nodes - mu) * lax.rsqrt(var + _LN_EPS) * lns_r[:] + lnb_r[:]
    h = _silu(_dotb(normed.astype(bf), nw1a_r[:])
              + _dotb(mi.astype(bf), nw1b_r[:]) + nb1_r[:])
    node_out = _dotb(h.astype(bf), nw2_r[:]) + nb2_r[:] + nodes
    gate = 0.5 * jnp.tanh(
        0.5 * (_dotb(node_out.astype(bf), gw_r[:]) + gb_r[:])) + 0.5

    nf = ht48 + upd48
    on_r[0] = node_out
    ofx_r[0] = nf[:, :_D1] * gate
    ofy_r[0] = nf[:, _D1:2 * _D1] * gate
    ofz_r[0] = nf[:, 2 * _D1:] * gate


def _pack_halves(x):
    u = lax.bitcast_convert_type(
        x.astype(jnp.bfloat16), jnp.uint16).astype(jnp.uint32)
    n = x.shape[-1] // 2
    return lax.bitcast_convert_type(
        u[..., :n] | (u[..., n:] << 16), jnp.int32)


def kernel(node_feats, htype1, rel_dist, neighbor_indices, neighbor_masks,
           ln_scale, ln_bias, e_w1, e_b1, e_w2, e_b2, hn_scale, hn_bias,
           gate_w, gate_b, ht_w1, ht_b1, ht_w2, ht_b2, n_w1, n_b1, n_w2, n_b2):
    f32 = jnp.float32
    bf = jnp.bfloat16
    nodes = node_feats[..., 0]
    ht48 = jnp.concatenate(
        [htype1[..., 0], htype1[..., 1], htype1[..., 2]],
        axis=-1)

    tblw = jnp.concatenate(
        [_pack_halves(nodes), _pack_halves(ht48),
         jnp.zeros((_B, _N, _GW - _D0 // 2 - 3 * _D1 // 2), jnp.int32)],
        axis=-1).reshape(_B * _N, _GW)
    fidx = (jnp.arange(_B, dtype=jnp.int32)[:, None, None] * _N
            + neighbor_indices.astype(jnp.int32)).reshape(_E)
    g = _gather_rows(tblw, fidx).reshape(_B, _N * _K, _GW)

    lane16 = jnp.arange(_D1, dtype=jnp.int32)
    aux = (rel_dist[..., None] * (lane16 == 0)
           + neighbor_masks[..., None].astype(f32) * (lane16 == 1)
           ).reshape(_B, _N * _K, _D1)

    wa = e_w1[:_D0].astype(bf)
    wbm = e_w1[_D0:2 * _D0]
    wb = wbm.astype(bf)
    wc = e_w1[2 * _D0:2 * _D0 + _D1].astype(bf)
    wdp = jnp.concatenate(
        [e_w1[2 * _D0 + _D1:], jnp.zeros((_D1 - 1, _F1), f32)],
        axis=0).astype(bf)
    sel = (jnp.arange(_R, dtype=jnp.int32)[:, None] // _K
           == jnp.arange(_BN, dtype=jnp.int32)[None, :]).astype(bf)
    nw1a = n_w1[:_D0].astype(bf)
    nw1b = n_w1[_D0:].astype(bf)
    row = lambda v: v.reshape(1, -1)

    grid = (_B, _N // _BN)
    node_spec = pl.BlockSpec((1, _BN, _D0), lambda b, i: (b, i, 0))
    d1_spec = pl.BlockSpec((1, _BN, _D1), lambda b, i: (b, i, 0))
    ht_spec = pl.BlockSpec((1, _BN, 3 * _D1), lambda b, i: (b, i, 0))
    g_spec = pl.BlockSpec((1, _R, _GW), lambda b, i: (b, i, 0))
    aux_spec = pl.BlockSpec((1, _R, _D1), lambda b, i: (b, i, 0))
    _full = lambda shape: pl.BlockSpec(shape, lambda b, i: (0, 0))

    on, ofx, ofy, ofz = pl.pallas_call(
        _tc_body,
        grid=grid,
        in_specs=[
            node_spec, ht_spec, g_spec, aux_spec,
            _full((_R, _BN)),
            _full((_D0, _F1)), _full((_D0, _F1)), _full((_D1, _F1)),
            _full((_D1, _F1)), _full((1, _F1)), _full((_F1, _H)),
            _full((1, _H)),
            _full((_H, _F2)), _full((1, _F2)), _full((_F2, _D1)),
            _full((1, _D1)),
            _full((_D0, 2 * _D0)), _full((_H, 2 * _D0)), _full((1, 2 * _D0)),
            _full((2 * _D0, _D0)), _full((1, _D0)),
            _full((_D0, _D1)), _full((1, _D1)), _full((1, _D0)),
            _full((1, _D0)), _full((1, _D1)), _full((1, _D1)),
        ],
        out_specs=[node_spec, d1_spec, d1_spec, d1_spec],
        out_shape=[
            jax.ShapeDtypeStruct((_B, _N, _D0), f32),
            jax.ShapeDtypeStruct((_B, _N, _D1), f32),
            jax.ShapeDtypeStruct((_B, _N, _D1), f32),
            jax.ShapeDtypeStruct((_B, _N, _D1), f32),
        ],
    )(nodes, ht48, g, aux, sel,
      wa, wb, wc, wdp, row(e_b1), e_w2.astype(bf), row(e_b2),
      ht_w1.astype(bf), row(ht_b1), ht_w2.astype(bf), row(ht_b2),
      nw1a, nw1b, row(n_b1), n_w2.astype(bf), row(n_b2),
      gate_w.astype(bf), row(gate_b), row(ln_scale), row(ln_bias),
      hn_scale.reshape(1, _D1), hn_bias.reshape(1, _D1))

    node_out = on[..., None]
    feat1 = jnp.stack([ofx, ofy, ofz], axis=-1)
    return node_out, feat1
